# baseline (speedup 1.0000x reference)
import functools

import jax
import jax.numpy as jnp
from jax import lax
from jax.experimental import pallas as pl
from jax.experimental.pallas import tpu as pltpu
from jax.experimental.pallas import tpu_sc as plsc

MAXLEN = 200
EMBED_DIM = 64
LANES = 16

NUM_WORKERS = 32
SEQ_PER_WORKER = 128
SEQ_PER_CHUNK = 4
CHUNK_ROWS = SEQ_PER_CHUNK * MAXLEN
CHUNKS = SEQ_PER_WORKER // SEQ_PER_CHUNK
SUB = 100
NSUB = CHUNK_ROWS // SUB


def _emb_body(x_hbm, tok_hbm, pos_hbm, out_hbm, idx_v, rows_v, pos_v, gsem):
    wid = lax.axis_index("s") * 2 + lax.axis_index("c")
    base_row = wid * (SEQ_PER_WORKER * MAXLEN)

    pltpu.sync_copy(pos_hbm, pos_v)

    def chunk_body(g, carry):
        base = pl.multiple_of(base_row + g * CHUNK_ROWS, CHUNK_ROWS)
        pltpu.sync_copy(x_hbm.at[pl.ds(pl.multiple_of(base // SUB, 8), NSUB)], idx_v)
        descs = []
        for i in range(NSUB):
            descs.append(
                pltpu.async_copy(
                    tok_hbm.at[idx_v.at[i]],
                    rows_v.at[pl.ds(i * SUB, SUB)],
                    gsem,
                )
            )
        for d in descs:
            d.wait()

        def pos_body(t, c):
            for j in range(EMBED_DIM // LANES):
                pv = pos_v[t, pl.ds(j * LANES, LANES)]
                for s in range(SEQ_PER_CHUNK):
                    plsc.addupdate(
                        rows_v.at[s * MAXLEN + t, pl.ds(j * LANES, LANES)], pv
                    )
            return c

        lax.fori_loop(0, MAXLEN, pos_body, None)

        pltpu.sync_copy(rows_v, out_hbm.at[pl.ds(base, CHUNK_ROWS)])
        return carry

    lax.fori_loop(0, CHUNKS, chunk_body, None)


def kernel(x, token_table, pos_table):
    batch, maxlen = x.shape
    _, embed_dim = token_table.shape
    n_rows = batch * maxlen
    x2 = x.reshape(n_rows // SUB, SUB).astype(jnp.int32)

    call = pl.kernel(
        _emb_body,
        out_type=jax.ShapeDtypeStruct((n_rows, embed_dim), jnp.float32),
        mesh=plsc.VectorSubcoreMesh(core_axis_name="c", subcore_axis_name="s"),
        scratch_types=[
            pltpu.VMEM((NSUB, SUB), jnp.int32),
            pltpu.VMEM((CHUNK_ROWS, EMBED_DIM), jnp.float32),
            pltpu.VMEM((MAXLEN, EMBED_DIM), jnp.float32),
            pltpu.SemaphoreType.DMA,
        ],
        compiler_params=pltpu.CompilerParams(use_tc_tiling_on_sc=False),
    )
    out_flat = call(x2, token_table, pos_table)
    return out_flat.reshape(batch, maxlen, embed_dim)


# speedup vs baseline: 1.0812x; 1.0812x over previous
import jax
import jax.numpy as jnp
from jax import lax
from jax.experimental import pallas as pl
from jax.experimental.pallas import tpu as pltpu
from jax.experimental.pallas import tpu_sc as plsc

MAXLEN = 200
EMBED_DIM = 64
LANES = 16

NUM_WORKERS = 32
SEQ_PER_WORKER = 128
SEQ_PER_CHUNK = 4
CHUNK_ROWS = SEQ_PER_CHUNK * MAXLEN
CHUNKS = SEQ_PER_WORKER // SEQ_PER_CHUNK
SUB = 100
NSUB = CHUNK_ROWS // SUB


def _emb_body(x_hbm, tok_hbm, pos_hbm, out_hbm, idx_v, rows_v, pos_v,
              gsems, osems, isems):
    wid = lax.axis_index("s") * 2 + lax.axis_index("c")
    base_row = wid * (SEQ_PER_WORKER * MAXLEN)

    def idx_load(c):
        b = c & 1
        return pltpu.async_copy(
            x_hbm.at[pl.ds(pl.multiple_of((base_row + c * CHUNK_ROWS) // SUB, 8),
                           NSUB)],
            idx_v.at[b], isems.at[b])

    def gathers(c):
        b = c & 1
        return [
            pltpu.async_copy(tok_hbm.at[idx_v.at[b, i]],
                             rows_v.at[b, pl.ds(i * SUB, SUB)], gsems.at[b])
            for i in range(NSUB)
        ]

    def out_write(c):
        b = c & 1
        return pltpu.async_copy(
            rows_v.at[b],
            out_hbm.at[pl.ds(pl.multiple_of(base_row + c * CHUNK_ROWS,
                                            CHUNK_ROWS), CHUNK_ROWS)],
            osems.at[b])

    idx_load(0).wait()
    gd = {0: gathers(0)}
    idd = {1: idx_load(1)}
    od = {}
    pltpu.sync_copy(pos_hbm, pos_v)

    for c in range(CHUNKS):
        b = c & 1
        for d in gd.pop(c):
            d.wait()
        if c + 2 < CHUNKS:
            idd[c + 2] = idx_load(c + 2)
        if c - 1 in od:
            od.pop(c - 1).wait()
        if c + 1 < CHUNKS:
            idd.pop(c + 1).wait()
            gd[c + 1] = gathers(c + 1)

        def pos_body(t, carry, _b=b):
            for j in range(EMBED_DIM // LANES):
                pv = pos_v[t, pl.ds(j * LANES, LANES)]
                for s in range(SEQ_PER_CHUNK):
                    plsc.addupdate(
                        rows_v.at[_b, s * MAXLEN + t, pl.ds(j * LANES, LANES)],
                        pv)
            return carry

        lax.fori_loop(0, MAXLEN, pos_body, None)
        od[c] = out_write(c)

    for c in list(od):
        od.pop(c).wait()


def kernel(x, token_table, pos_table):
    batch, maxlen = x.shape
    _, embed_dim = token_table.shape
    n_rows = batch * maxlen
    x2 = x.reshape(n_rows // SUB, SUB).astype(jnp.int32)

    call = pl.kernel(
        _emb_body,
        out_type=jax.ShapeDtypeStruct((n_rows, embed_dim), jnp.float32),
        mesh=plsc.VectorSubcoreMesh(core_axis_name="c", subcore_axis_name="s"),
        scratch_types=[
            pltpu.VMEM((2, NSUB, SUB), jnp.int32),
            pltpu.VMEM((2, CHUNK_ROWS, EMBED_DIM), jnp.float32),
            pltpu.VMEM((MAXLEN, EMBED_DIM), jnp.float32),
            pltpu.SemaphoreType.DMA((2,)),
            pltpu.SemaphoreType.DMA((2,)),
            pltpu.SemaphoreType.DMA((2,)),
        ],
        compiler_params=pltpu.CompilerParams(use_tc_tiling_on_sc=False),
    )
    out_flat = call(x2, token_table, pos_table)
    return out_flat.reshape(batch, maxlen, embed_dim)


# speedup vs baseline: 1.3074x; 1.2093x over previous
import jax
import jax.numpy as jnp
from jax import lax
from jax.experimental import pallas as pl
from jax.experimental.pallas import tpu as pltpu
from jax.experimental.pallas import tpu_sc as plsc

MAXLEN = 200
EMBED_DIM = 64
PAD_DIM = 128
LANES = 16

NUM_WORKERS = 32
SEQ_PER_WORKER = 128
CHUNK_ROWS = MAXLEN
CHUNKS = SEQ_PER_WORKER
SUB = 100
NSUB = CHUNK_ROWS // SUB
IDX_ROWS = SEQ_PER_WORKER * MAXLEN // SUB


def _emb_body(x_hbm, tok_hbm, pos_hbm, out_hbm, idx_v, rows_v, pos_v,
              gsems, osems):
    wid = lax.axis_index("s") * 2 + lax.axis_index("c")
    base_row = wid * (SEQ_PER_WORKER * MAXLEN)

    pltpu.sync_copy(
        x_hbm.at[pl.ds(pl.multiple_of(wid * IDX_ROWS, 8), IDX_ROWS)], idx_v)
    pltpu.sync_copy(pos_hbm, pos_v)

    def gather(c, b, issue):
        for i in range(NSUB):
            d = pltpu.make_async_copy(
                tok_hbm.at[idx_v.at[NSUB * c + i]],
                rows_v.at[b, pl.ds(i * SUB, SUB)],
                gsems.at[b])
            d.start() if issue else d.wait()

    def out_write(c, b, issue):
        d = pltpu.make_async_copy(
            rows_v.at[b],
            out_hbm.at[pl.ds(pl.multiple_of(base_row + c * CHUNK_ROWS, 8),
                             CHUNK_ROWS)],
            osems.at[b])
        d.start() if issue else d.wait()

    gather(0, 0, True)

    def pair_body(g, carry):
        for b in range(2):
            c = 2 * g + b
            gather(c, b, False)
            @pl.when(c + 1 < CHUNKS)
            def _():
                @pl.when(c >= 1)
                def _():
                    out_write(c - 1, 1 - b, False)
                gather(c + 1, 1 - b, True)

            def pos_body(t, carry2, _b=b):
                for j in range(EMBED_DIM // LANES):
                    pv = pos_v[t, pl.ds(j * LANES, LANES)]
                    plsc.addupdate(
                        rows_v.at[_b, t, pl.ds(j * LANES, LANES)], pv)
                return carry2

            lax.fori_loop(0, MAXLEN, pos_body, None)
            out_write(c, b, True)
        return carry

    lax.fori_loop(0, CHUNKS // 2, pair_body, None)
    out_write(CHUNKS - 2, 0, False)
    out_write(CHUNKS - 1, 1, False)


def kernel(x, token_table, pos_table):
    batch, maxlen = x.shape
    _, embed_dim = token_table.shape
    n_rows = batch * maxlen
    x2 = x.reshape(n_rows // SUB, SUB).astype(jnp.int32)
    tok_pad = jnp.pad(token_table, ((0, 0), (0, PAD_DIM - embed_dim)))

    call = pl.kernel(
        _emb_body,
        out_type=jax.ShapeDtypeStruct((n_rows, PAD_DIM), jnp.float32),
        mesh=plsc.VectorSubcoreMesh(core_axis_name="c", subcore_axis_name="s"),
        scratch_types=[
            pltpu.VMEM((IDX_ROWS, SUB), jnp.int32),
            pltpu.VMEM((2, CHUNK_ROWS, PAD_DIM), jnp.float32),
            pltpu.VMEM((MAXLEN, EMBED_DIM), jnp.float32),
            pltpu.SemaphoreType.DMA((2,)),
            pltpu.SemaphoreType.DMA((2,)),
        ],
        compiler_params=pltpu.CompilerParams(use_tc_tiling_on_sc=True),
    )
    out_flat = call(x2, tok_pad, pos_table)
    return out_flat[:, :embed_dim].reshape(batch, maxlen, embed_dim)


# speedup vs baseline: 1.6039x; 1.2268x over previous
import jax
import jax.numpy as jnp
from jax import lax
from jax.experimental import pallas as pl
from jax.experimental.pallas import tpu as pltpu
from jax.experimental.pallas import tpu_sc as plsc

MAXLEN = 200
EMBED_DIM = 64
PAD_DIM = 128
LANES = 16

NUM_WORKERS = 32
SEQ_PER_WORKER = 128
CHUNK_ROWS = MAXLEN
CHUNKS = SEQ_PER_WORKER
SUB = 100
NSUB = CHUNK_ROWS // SUB
IDX_ROWS = SEQ_PER_WORKER * MAXLEN // SUB


def _emb_body(x_hbm, tok_hbm, pos_hbm, out_hbm, idx_v, rows_v, pos_v,
              gsems, osems):
    wid = lax.axis_index("s") * 2 + lax.axis_index("c")
    base_row = wid * (SEQ_PER_WORKER * MAXLEN)

    pltpu.sync_copy(
        x_hbm.at[pl.ds(pl.multiple_of(wid * IDX_ROWS, 8), IDX_ROWS)], idx_v)
    pltpu.sync_copy(pos_hbm, pos_v)

    def gather(c, b, issue):
        for i in range(NSUB):
            d = pltpu.make_async_copy(
                tok_hbm.at[idx_v.at[NSUB * c + i]],
                rows_v.at[b, pl.ds(i * SUB, SUB)],
                gsems.at[b])
            d.start() if issue else d.wait()

    def out_write(c, b, issue):
        d = pltpu.make_async_copy(
            rows_v.at[b],
            out_hbm.at[pl.ds(pl.multiple_of(base_row + c * CHUNK_ROWS, 8),
                             CHUNK_ROWS)],
            osems.at[b])
        d.start() if issue else d.wait()

    gather(0, 0, True)

    def pair_body(g, carry):
        for b in range(2):
            c = 2 * g + b
            gather(c, b, False)
            @pl.when(c + 1 < CHUNKS)
            def _():
                @pl.when(c >= 1)
                def _():
                    out_write(c - 1, 1 - b, False)
                gather(c + 1, 1 - b, True)

            def pos_body(t, carry2, _b=b):
                for j in range(EMBED_DIM // LANES):
                    pv = pos_v[t, pl.ds(j * LANES, LANES)]
                    plsc.addupdate(
                        rows_v.at[_b, t, pl.ds(j * LANES, LANES)], pv)
                return carry2

            lax.fori_loop(0, MAXLEN, pos_body, None)
            out_write(c, b, True)
        return carry

    lax.fori_loop(0, CHUNKS // 2, pair_body, None)
    out_write(CHUNKS - 2, 0, False)
    out_write(CHUNKS - 1, 1, False)


def _transpose_pad_body(xt_ref, o_ref):
    o_ref[:, :EMBED_DIM] = xt_ref[...].T
    o_ref[:, EMBED_DIM:] = jnp.zeros_like(o_ref[:, EMBED_DIM:])


def _transpose_pad_table(tok_t):
    _, vocab = tok_t.shape
    blk = 4096
    return pl.pallas_call(
        _transpose_pad_body,
        grid=(vocab // blk,),
        in_specs=[pl.BlockSpec((EMBED_DIM, blk), lambda i: (0, i))],
        out_specs=pl.BlockSpec((blk, PAD_DIM), lambda i: (i, 0)),
        out_shape=jax.ShapeDtypeStruct((vocab, PAD_DIM), jnp.float32),
    )(tok_t)


def kernel(x, token_table, pos_table):
    batch, maxlen = x.shape
    _, embed_dim = token_table.shape
    n_rows = batch * maxlen
    x2 = x.reshape(n_rows // SUB, SUB).astype(jnp.int32)
    tok_pad = _transpose_pad_table(token_table.T)

    call = pl.kernel(
        _emb_body,
        out_type=jax.ShapeDtypeStruct((n_rows, PAD_DIM), jnp.float32),
        mesh=plsc.VectorSubcoreMesh(core_axis_name="c", subcore_axis_name="s"),
        scratch_types=[
            pltpu.VMEM((IDX_ROWS, SUB), jnp.int32),
            pltpu.VMEM((2, CHUNK_ROWS, PAD_DIM), jnp.float32),
            pltpu.VMEM((MAXLEN, EMBED_DIM), jnp.float32),
            pltpu.SemaphoreType.DMA((2,)),
            pltpu.SemaphoreType.DMA((2,)),
        ],
        compiler_params=pltpu.CompilerParams(use_tc_tiling_on_sc=True),
    )
    out_flat = call(x2, tok_pad, pos_table)
    return out_flat[:, :embed_dim].reshape(batch, maxlen, embed_dim)


# speedup vs baseline: 1.7453x; 1.0881x over previous
import jax
import jax.numpy as jnp
from jax import lax
from jax.experimental import pallas as pl
from jax.experimental.pallas import tpu as pltpu
from jax.experimental.pallas import tpu_sc as plsc

MAXLEN = 200
EMBED_DIM = 64
PAD_DIM = 128
LANES = 16

NUM_WORKERS = 32
SEQ_PER_WORKER = 128
CHUNK_ROWS = MAXLEN
CHUNKS = SEQ_PER_WORKER
SUB = 100
NSUB = CHUNK_ROWS // SUB
IDX_ROWS = SEQ_PER_WORKER * MAXLEN // SUB


def _emb_body(x_hbm, tok_hbm, pos_hbm, out_hbm, idx_v, rows_v, pos_v,
              gsems, osems):
    wid = lax.axis_index("s") * 2 + lax.axis_index("c")
    base_row = wid * (SEQ_PER_WORKER * MAXLEN)

    pltpu.sync_copy(
        x_hbm.at[pl.ds(pl.multiple_of(wid * IDX_ROWS, 8), IDX_ROWS)], idx_v)
    pltpu.sync_copy(pos_hbm, pos_v)

    def gather(c, b, issue):
        for i in range(NSUB):
            d = pltpu.make_async_copy(
                tok_hbm.at[idx_v.at[NSUB * c + i]],
                rows_v.at[b, pl.ds(i * SUB, SUB)],
                gsems.at[b])
            d.start() if issue else d.wait()

    def out_write(c, b, issue):
        d = pltpu.make_async_copy(
            rows_v.at[b],
            out_hbm.at[pl.ds(pl.multiple_of(base_row + c * CHUNK_ROWS, 8),
                             CHUNK_ROWS)],
            osems.at[b])
        d.start() if issue else d.wait()

    gather(0, 0, True)

    def pair_body(g, carry):
        for b in range(2):
            c = 2 * g + b
            gather(c, b, False)
            @pl.when(c + 1 < CHUNKS)
            def _():
                @pl.when(c >= 1)
                def _():
                    out_write(c - 1, 1 - b, False)
                gather(c + 1, 1 - b, True)

            def pos_body(t, carry2, _b=b):
                for j in range(EMBED_DIM // LANES):
                    pv = pos_v[t, pl.ds(j * LANES, LANES)]
                    plsc.addupdate(
                        rows_v.at[_b, t, pl.ds(j * LANES, LANES)], pv)
                return carry2

            lax.fori_loop(0, MAXLEN, pos_body, None)
            out_write(c, b, True)
        return carry

    lax.fori_loop(0, CHUNKS // 2, pair_body, None)
    out_write(CHUNKS - 2, 0, False)
    out_write(CHUNKS - 1, 1, False)


def _transpose_pad_body(xt_ref, o_ref):
    o_ref[:, :EMBED_DIM] = xt_ref[...].T
    o_ref[:, EMBED_DIM:] = jnp.zeros_like(o_ref[:, EMBED_DIM:])


def _transpose_pad_table(tok_t):
    _, vocab = tok_t.shape
    blk = 7936
    return pl.pallas_call(
        _transpose_pad_body,
        grid=(pl.cdiv(vocab, blk),),
        in_specs=[pl.BlockSpec((EMBED_DIM, blk), lambda i: (0, i))],
        out_specs=pl.BlockSpec((blk, PAD_DIM), lambda i: (i, 0)),
        out_shape=jax.ShapeDtypeStruct((vocab, PAD_DIM), jnp.float32),
    )(tok_t)


def kernel(x, token_table, pos_table):
    batch, maxlen = x.shape
    _, embed_dim = token_table.shape
    n_rows = batch * maxlen
    x2 = x.reshape(n_rows // SUB, SUB).astype(jnp.int32)
    tok_pad = _transpose_pad_table(token_table.T)

    call = pl.kernel(
        _emb_body,
        out_type=jax.ShapeDtypeStruct((n_rows, PAD_DIM), jnp.float32),
        mesh=plsc.VectorSubcoreMesh(core_axis_name="c", subcore_axis_name="s"),
        scratch_types=[
            pltpu.VMEM((IDX_ROWS, SUB), jnp.int32),
            pltpu.VMEM((2, CHUNK_ROWS, PAD_DIM), jnp.float32),
            pltpu.VMEM((MAXLEN, EMBED_DIM), jnp.float32),
            pltpu.SemaphoreType.DMA((2,)),
            pltpu.SemaphoreType.DMA((2,)),
        ],
        compiler_params=pltpu.CompilerParams(use_tc_tiling_on_sc=True),
    )
    out_flat = call(x2, tok_pad, pos_table)
    return out_flat[:, :embed_dim].reshape(batch, maxlen, embed_dim)
